# baseline (speedup 1.0000x reference)
import jax
import jax.numpy as jnp
from jax.experimental import pallas as pl
from jax.experimental.pallas import tpu as pltpu

_N = 10000
_D = 128
_BM = 400
_BH = _BM // 2
_NB = _N // _BM


def _fused_kernel(x_ref, adj_a_ref, adj_b_ref, w1_ref, b1_ref, w2_ref,
                  b2_ref, w3_ref, b3_ref, w4_ref, b4_ref,
                  emb_ref, z_ref, g1_ref, g2_ref):
    i = pl.program_id(0)

    @pl.when(i == 0)
    def _g1_phase():
        acc = jnp.dot(x_ref[...], w1_ref[...],
                      preferred_element_type=jnp.float32) + b1_ref[...]
        g1_ref[...] = acc.astype(jnp.bfloat16)

    @pl.when((i >= 1) & (i <= _NB))
    def _pass1_phase():
        for k, aref in enumerate((adj_a_ref, adj_b_ref)):
            a = aref[...].astype(jnp.bfloat16)
            h = jnp.dot(a, g1_ref[...], preferred_element_type=jnp.float32)
            h = jnp.maximum(h, 0.0)
            g2 = jnp.dot(h, w2_ref[...],
                         preferred_element_type=jnp.float32) + b2_ref[...]
            g2_ref[pl.ds((i - 1) * _BM + k * _BH, _BH), :] = (
                g2.astype(jnp.bfloat16))

    @pl.when(i > _NB)
    def _pass2_phase():
        for k, aref in enumerate((adj_a_ref, adj_b_ref)):
            a = aref[...].astype(jnp.bfloat16)
            emb = jnp.dot(a, g2_ref[...], preferred_element_type=jnp.float32)
            emb_ref[k * _BH:(k + 1) * _BH, :] = emb
            t = jnp.dot(emb, w3_ref[...],
                        preferred_element_type=jnp.float32) + b3_ref[...]
            t = jnp.maximum(t, 0.0)
            z_ref[k * _BH:(k + 1) * _BH, :] = jnp.dot(
                t, w4_ref[...], preferred_element_type=jnp.float32) + b4_ref[...]


def _blk(i):
    return jnp.where(i <= _NB, jnp.maximum(i - 1, 0), i - 1 - _NB)


def _adj_a_map(i):
    return (2 * _blk(i), 0)


def _adj_b_map(i):
    return (2 * _blk(i) + 1, 0)


def _out_map(i):
    return (jnp.clip(i - 1 - _NB, 0, _NB - 1), 0)


def _const_map(i):
    return (0, 0)


def kernel(x, Adj_, W1, b1, W2, b2, W3, b3, W4, b4):
    full = lambda r, c: pl.BlockSpec((r, c), _const_map)
    emb, z = pl.pallas_call(
        _fused_kernel,
        grid=(1 + 2 * _NB,),
        in_specs=[
            full(_N, _D),
            pl.BlockSpec((_BH, _N), _adj_a_map),
            pl.BlockSpec((_BH, _N), _adj_b_map),
            full(_D, _D), full(1, _D),
            full(_D, _D), full(1, _D),
            full(_D, _D), full(1, _D),
            full(_D, _D), full(1, _D),
        ],
        out_specs=[
            pl.BlockSpec((_BM, _D), _out_map),
            pl.BlockSpec((_BM, _D), _out_map),
        ],
        out_shape=[
            jax.ShapeDtypeStruct((_N, _D), jnp.float32),
            jax.ShapeDtypeStruct((_N, _D), jnp.float32),
        ],
        scratch_shapes=[
            pltpu.VMEM((_N, _D), jnp.bfloat16),
            pltpu.VMEM((_N, _D), jnp.bfloat16),
        ],
    )(x, Adj_, Adj_, W1, b1.reshape(1, _D), W2, b2.reshape(1, _D),
      W3, b3.reshape(1, _D), W4, b4.reshape(1, _D))
    return (z, emb)
